# baseline (speedup 1.0000x reference)
import functools

import jax
import jax.numpy as jnp
from jax import lax
from jax.experimental import pallas as pl
from jax.experimental.pallas import tpu as pltpu
from jax.experimental.pallas import tpu_sc as plsc

_THETA = (0.5, 0.25, 0.125, 0.0625, 0.03125)
_N = 10000
_D = 128
_NE = 320000
_NCORES = 2
_NSUB = 16
_NW = _NCORES * _NSUB
_CHUNK = 64
_CPW = 160
_NE_PAD = _NW * _CPW * _CHUNK
_NPAD = 10240
_RPT = _NPAD // _NSUB
_NROUND = _CPW // 2


def _make_spmm():
    mesh = plsc.VectorSubcoreMesh(core_axis_name="c", subcore_axis_name="s")

    @functools.partial(
        pl.kernel,
        out_type=jax.ShapeDtypeStruct((_NCORES, _NPAD, _D), jnp.float32),
        mesh=mesh,
        scratch_types=[
            pltpu.VMEM((_NROUND, _CHUNK * 2), jnp.int32),
            pltpu.VMEM((_NROUND, _CHUNK * 2), jnp.int32),
            pltpu.VMEM((_NROUND, _CHUNK * 2), jnp.float32),
            pltpu.VMEM((_CHUNK, _D), jnp.float32),
            pltpu.VMEM((_CHUNK, _D), jnp.float32),
            pltpu.VMEM_SHARED((_NPAD, _D), jnp.float32),
            pltpu.SemaphoreType.DMA,
            pltpu.SemaphoreType.DMA,
        ],
    )
    def spmm(src_hbm, dst_hbm, w_hbm, f_hbm, zeros_hbm, out_hbm,
             src_v, dst_v, w_v, rows0_v, rows1_v, acc_sh, sem0, sem1):
        c = lax.axis_index("c")
        s = lax.axis_index("s")
        wid = c * _NSUB + s
        row0 = s * _RPT
        pltpu.sync_copy(zeros_hbm.at[pl.ds(row0, _RPT)],
                        acc_sh.at[pl.ds(row0, _RPT)])
        pltpu.sync_copy(src_hbm.at[wid], src_v)
        pltpu.sync_copy(dst_hbm.at[wid], dst_v)
        pltpu.sync_copy(w_hbm.at[wid], w_v)
        plsc.subcore_barrier()

        def gather(r, b, rows, sem):
            pltpu.async_copy(
                f_hbm.at[src_v.at[r, pl.ds(b * _CHUNK, _CHUNK)]], rows, sem)

        def gather_wait(rows, sem):
            pltpu.make_async_copy(f_hbm.at[pl.ds(0, _CHUNK)], rows, sem).wait()

        def scale(rows, r, b):
            def group_body(g, carry2):
                wv16 = w_v[r, pl.ds(b * _CHUNK + g * 16, 16)]
                base = g * 16
                for e16 in range(16):
                    wv = wv16[e16]
                    for t in range(_D // 16):
                        sl = pl.ds(t * 16, 16)
                        rows[base + e16, sl] = rows[base + e16, sl] * wv
                return carry2

            lax.fori_loop(0, _CHUNK // 16, group_body, 0)

        gather(0, 0, rows0_v, sem0)
        gather(0, 1, rows1_v, sem1)

        def round_body(r, carry):
            for b, rows, sem in ((0, rows0_v, sem0), (1, rows1_v, sem1)):
                gather_wait(rows, sem)
                scale(rows, r, b)
                pltpu.sync_copy(
                    rows, acc_sh.at[dst_v.at[r, pl.ds(b * _CHUNK, _CHUNK)]],
                    add=True)

                @pl.when(r < _NROUND - 1)
                def _():
                    gather(r + 1, b, rows, sem)
            return carry

        lax.fori_loop(0, _NROUND, round_body, 0)
        plsc.subcore_barrier()
        pltpu.sync_copy(acc_sh.at[pl.ds(row0, _RPT)],
                        out_hbm.at[c, pl.ds(row0, _RPT)])

    return spmm


_spmm = _make_spmm()


def _make_combine(theta, first):
    def body(f_ref, p0_ref, p1_ref, h_ref, fo_ref, ho_ref):
        fn = f_ref[...] - (p0_ref[...] + p1_ref[...])
        fo_ref[...] = fn
        if first:
            ho_ref[...] = _THETA[0] * f_ref[...] + theta * fn
        else:
            ho_ref[...] = h_ref[...] + theta * fn

    blk = pl.BlockSpec((1000, _D), lambda i: (i, 0))
    out_sd = jax.ShapeDtypeStruct((_N, _D), jnp.float32)
    return pl.pallas_call(
        body,
        grid=(_N // 1000,),
        in_specs=[blk, blk, blk, blk],
        out_specs=[blk, blk],
        out_shape=[out_sd, out_sd],
    )


_combine = [_make_combine(_THETA[k], first=(k == 1)) for k in range(1, 5)]


def kernel(norm_adj_edge_index, norm_adj_edge_weight, feat):
    src = norm_adj_edge_index[0].astype(jnp.int32)
    dst = norm_adj_edge_index[1].astype(jnp.int32)
    w = norm_adj_edge_weight.astype(jnp.float32)
    order = jnp.argsort(src)
    src = src[order]
    dst = dst[order]
    w = w[order]
    pad = _NE_PAD - _NE
    src_p = jnp.concatenate([src, jnp.zeros((pad,), jnp.int32)])
    dst_p = jnp.concatenate([dst, jnp.zeros((pad,), jnp.int32)])
    w_p = jnp.concatenate([w, jnp.zeros((pad,), jnp.float32)])
    src_p = src_p.reshape(_NW, _NROUND, _CHUNK * 2)
    dst_p = dst_p.reshape(_NW, _NROUND, _CHUNK * 2)
    w_p = w_p.reshape(_NW, _NROUND, _CHUNK * 2)
    zeros = jnp.zeros((_NPAD, _D), jnp.float32)

    f = feat
    h = feat
    for k in range(1, 5):
        part = _spmm(src_p, dst_p, w_p, f, zeros)
        f, h = _combine[k - 1](f, part[0, :_N], part[1, :_N], h)
    return h


# speedup vs baseline: 1.2558x; 1.2558x over previous
import functools

import jax
import jax.numpy as jnp
from jax import lax
from jax.experimental import pallas as pl
from jax.experimental.pallas import tpu as pltpu
from jax.experimental.pallas import tpu_sc as plsc

_THETA = (0.5, 0.25, 0.125, 0.0625, 0.03125)
_N = 10000
_D = 128
_NE = 320000
_NCORES = 2
_NSUB = 16
_NW = _NCORES * _NSUB
_CHUNK = 32
_NBUF = 4
_CPW = 320
_NE_PAD = _NW * _CPW * _CHUNK
_NPAD = 10240
_RPT = _NPAD // _NSUB
_NROUND = _CPW // _NBUF


def _make_spmm():
    mesh = plsc.VectorSubcoreMesh(core_axis_name="c", subcore_axis_name="s")

    @functools.partial(
        pl.kernel,
        out_type=jax.ShapeDtypeStruct((_NCORES, _NPAD, _D), jnp.float32),
        mesh=mesh,
        scratch_types=[
            pltpu.VMEM((_NROUND, _CHUNK * _NBUF), jnp.int32),
            pltpu.VMEM((_NROUND, _CHUNK * _NBUF), jnp.int32),
            pltpu.VMEM((_NROUND, _CHUNK * _NBUF), jnp.float32),
            pltpu.VMEM((_CHUNK, _D), jnp.float32),
            pltpu.VMEM((_CHUNK, _D), jnp.float32),
            pltpu.VMEM((_CHUNK, _D), jnp.float32),
            pltpu.VMEM((_CHUNK, _D), jnp.float32),
            pltpu.VMEM_SHARED((_NPAD, _D), jnp.float32),
            pltpu.SemaphoreType.DMA,
            pltpu.SemaphoreType.DMA,
            pltpu.SemaphoreType.DMA,
            pltpu.SemaphoreType.DMA,
        ],
    )
    def spmm(src_hbm, dst_hbm, w_hbm, f_hbm, zeros_hbm, out_hbm,
             src_v, dst_v, w_v, rows0_v, rows1_v, rows2_v, rows3_v,
             acc_sh, sem0, sem1, sem2, sem3):
        c = lax.axis_index("c")
        s = lax.axis_index("s")
        wid = c * _NSUB + s
        row0 = s * _RPT
        pltpu.sync_copy(zeros_hbm.at[pl.ds(row0, _RPT)],
                        acc_sh.at[pl.ds(row0, _RPT)])
        pltpu.sync_copy(src_hbm.at[wid], src_v)
        pltpu.sync_copy(dst_hbm.at[wid], dst_v)
        pltpu.sync_copy(w_hbm.at[wid], w_v)
        plsc.subcore_barrier()

        def gather(r, b, rows, sem):
            pltpu.async_copy(
                f_hbm.at[src_v.at[r, pl.ds(b * _CHUNK, _CHUNK)]], rows, sem)

        def gather_wait(rows, sem):
            pltpu.make_async_copy(f_hbm.at[pl.ds(0, _CHUNK)], rows, sem).wait()

        def scale(rows, r, b):
            def group_body(g, carry2):
                wv16 = w_v[r, pl.ds(b * _CHUNK + g * 16, 16)]
                base = g * 16
                for e16 in range(16):
                    wv = wv16[e16]
                    for t in range(_D // 16):
                        sl = pl.ds(t * 16, 16)
                        rows[base + e16, sl] = rows[base + e16, sl] * wv
                return carry2

            lax.fori_loop(0, _CHUNK // 16, group_body, 0)

        bufs = ((0, rows0_v, sem0), (1, rows1_v, sem1),
                (2, rows2_v, sem2), (3, rows3_v, sem3))

        for b, rows, sem in bufs:
            gather(0, b, rows, sem)

        def round_body(r, carry):
            for b, rows, sem in bufs:
                gather_wait(rows, sem)
                scale(rows, r, b)
                pltpu.sync_copy(
                    rows, acc_sh.at[dst_v.at[r, pl.ds(b * _CHUNK, _CHUNK)]],
                    add=True)

                @pl.when(r < _NROUND - 1)
                def _():
                    gather(r + 1, b, rows, sem)
            return carry

        lax.fori_loop(0, _NROUND, round_body, 0)
        plsc.subcore_barrier()
        pltpu.sync_copy(acc_sh.at[pl.ds(row0, _RPT)],
                        out_hbm.at[c, pl.ds(row0, _RPT)])

    return spmm


_spmm = _make_spmm()


def _make_combine(theta, first):
    def body(f_ref, p0_ref, p1_ref, h_ref, fo_ref, ho_ref):
        fn = f_ref[...] - (p0_ref[...] + p1_ref[...])
        fo_ref[...] = fn
        if first:
            ho_ref[...] = _THETA[0] * f_ref[...] + theta * fn
        else:
            ho_ref[...] = h_ref[...] + theta * fn

    blk = pl.BlockSpec((1000, _D), lambda i: (i, 0))
    out_sd = jax.ShapeDtypeStruct((_N, _D), jnp.float32)
    return pl.pallas_call(
        body,
        grid=(_N // 1000,),
        in_specs=[blk, blk, blk, blk],
        out_specs=[blk, blk],
        out_shape=[out_sd, out_sd],
    )


_combine = [_make_combine(_THETA[k], first=(k == 1)) for k in range(1, 5)]


def kernel(norm_adj_edge_index, norm_adj_edge_weight, feat):
    src = norm_adj_edge_index[0].astype(jnp.int32)
    dst = norm_adj_edge_index[1].astype(jnp.int32)
    w = norm_adj_edge_weight.astype(jnp.float32)
    pad = _NE_PAD - _NE
    src_p = jnp.concatenate([src, jnp.zeros((pad,), jnp.int32)])
    dst_p = jnp.concatenate([dst, jnp.zeros((pad,), jnp.int32)])
    w_p = jnp.concatenate([w, jnp.zeros((pad,), jnp.float32)])
    src_p = src_p.reshape(_NW, _NROUND, _CHUNK * _NBUF)
    dst_p = dst_p.reshape(_NW, _NROUND, _CHUNK * _NBUF)
    w_p = w_p.reshape(_NW, _NROUND, _CHUNK * _NBUF)
    zeros = jnp.zeros((_NPAD, _D), jnp.float32)

    f = feat
    h = feat
    for k in range(1, 5):
        part = _spmm(src_p, dst_p, w_p, f, zeros)
        f, h = _combine[k - 1](f, part[0, :_N], part[1, :_N], h)
    return h
